# Initial kernel scaffold; baseline (speedup 1.0000x reference)
import functools

import jax
import jax.numpy as jnp
from jax import lax
from jax.experimental import pallas as pl
from jax.experimental.pallas import tpu as pltpu
from jax.experimental.pallas import tpu_sc as plsc

_VOCAB = 1000000
_DIM = 32
_BATCH = 16384
_TOTAL = 327680

_NC = 2
_NS = 16
_HALF = _DIM // 2
_CHUNK = 128
_PER_SUB = _TOTAL // _NS
_NCHUNK = _PER_SUB // _CHUNK
_SEG_PER_SUB = _BATCH // _NS


def _sc_embed(table2, idx2, segs, zeros):
    mesh = plsc.VectorSubcoreMesh(core_axis_name="c", subcore_axis_name="s")

    @functools.partial(
        pl.kernel,
        mesh=mesh,
        out_type=jax.ShapeDtypeStruct((_NC, _BATCH, _HALF), jnp.float32),
        scratch_types=[
            pltpu.VMEM((_NCHUNK, _CHUNK), jnp.int32),
            pltpu.VMEM((_NCHUNK, _CHUNK), jnp.int32),
            pltpu.VMEM((_CHUNK, _HALF), jnp.float32),
            pltpu.VMEM_SHARED((_BATCH, _HALF), jnp.float32),
            pltpu.SemaphoreType.DMA,
        ],
    )
    def k(table_hbm, idx_hbm, seg_hbm, zero_hbm, out_hbm,
          idx_v, seg_v, rows_v, acc_sh, sem):
        c = lax.axis_index("c")
        s = lax.axis_index("s")

        pltpu.sync_copy(zero_hbm.at[pl.ds(s * _SEG_PER_SUB, _SEG_PER_SUB)],
                        acc_sh.at[pl.ds(s * _SEG_PER_SUB, _SEG_PER_SUB)])
        pltpu.sync_copy(idx_hbm.at[c, s], idx_v)
        pltpu.sync_copy(seg_hbm.at[s], seg_v)
        plsc.subcore_barrier()

        def body(j, carry):
            pltpu.async_copy(table_hbm.at[idx_v.at[j]], rows_v, sem).wait()
            pltpu.sync_copy(rows_v, acc_sh.at[seg_v.at[j]], add=True)
            return carry

        lax.fori_loop(0, _NCHUNK, body, 0)
        plsc.subcore_barrier()

        pltpu.sync_copy(acc_sh.at[pl.ds(s * _SEG_PER_SUB, _SEG_PER_SUB)],
                        out_hbm.at[c, pl.ds(s * _SEG_PER_SUB, _SEG_PER_SUB)])

    return k(table2, idx2, segs, zeros)


def kernel(ids, segment_ids, table):
    ids = ids.astype(jnp.int32)
    segs = segment_ids.astype(jnp.int32).reshape(_NS, _NCHUNK, _CHUNK)
    table2 = table.reshape(_VOCAB * 2, _HALF)
    base = ids * 2
    idx2 = jnp.stack([base, base + 1]).reshape(_NC, _NS, _NCHUNK, _CHUNK)
    zeros = jnp.zeros((_BATCH, _HALF), jnp.float32)
    out = _sc_embed(table2, idx2, segs, zeros)
    out = out.transpose(1, 0, 2).reshape(_BATCH, _DIM)
    return jnp.expand_dims(out, axis=1)


# speedup vs baseline: 1.9961x; 1.9961x over previous
import functools

import jax
import jax.numpy as jnp
from jax import lax
from jax.experimental import pallas as pl
from jax.experimental.pallas import tpu as pltpu
from jax.experimental.pallas import tpu_sc as plsc

_VOCAB = 1000000
_DIM = 32
_BATCH = 16384
_TOTAL = 327680

_NC = 2
_NS = 16
_HALF = _DIM // 2
_CHUNK = 128
_PER_SUB = _TOTAL // _NS
_NCHUNK = _PER_SUB // _CHUNK
_SEG_PER_SUB = _BATCH // _NS


def _sc_embed(table2, idx2, segs, zeros):
    mesh = plsc.VectorSubcoreMesh(core_axis_name="c", subcore_axis_name="s")

    @functools.partial(
        pl.kernel,
        mesh=mesh,
        compiler_params=pltpu.CompilerParams(use_tc_tiling_on_sc=False),
        out_type=jax.ShapeDtypeStruct((_NC, _BATCH, _HALF), jnp.float32),
        scratch_types=[
            pltpu.VMEM((_NCHUNK, _CHUNK), jnp.int32),
            pltpu.VMEM((_NCHUNK, _CHUNK), jnp.int32),
            pltpu.VMEM((_CHUNK, _HALF), jnp.float32),
            pltpu.VMEM_SHARED((_BATCH, _HALF), jnp.float32),
            pltpu.SemaphoreType.DMA,
        ],
    )
    def k(table_hbm, idx_hbm, seg_hbm, zero_hbm, out_hbm,
          idx_v, seg_v, rows_v, acc_sh, sem):
        c = lax.axis_index("c")
        s = lax.axis_index("s")

        pltpu.sync_copy(zero_hbm.at[pl.ds(s * _SEG_PER_SUB, _SEG_PER_SUB)],
                        acc_sh.at[pl.ds(s * _SEG_PER_SUB, _SEG_PER_SUB)])
        pltpu.sync_copy(idx_hbm.at[c, s], idx_v)
        pltpu.sync_copy(seg_hbm.at[s], seg_v)
        plsc.subcore_barrier()

        def body(j, carry):
            pltpu.async_copy(table_hbm.at[idx_v.at[j]], rows_v, sem).wait()
            pltpu.sync_copy(rows_v, acc_sh.at[seg_v.at[j]], add=True)
            return carry

        lax.fori_loop(0, _NCHUNK, body, 0)
        plsc.subcore_barrier()

        pltpu.sync_copy(acc_sh.at[pl.ds(s * _SEG_PER_SUB, _SEG_PER_SUB)],
                        out_hbm.at[c, pl.ds(s * _SEG_PER_SUB, _SEG_PER_SUB)])

    return k(table2, idx2, segs, zeros)


def kernel(ids, segment_ids, table):
    ids = ids.astype(jnp.int32)
    segs = segment_ids.astype(jnp.int32).reshape(_NS, _NCHUNK, _CHUNK)
    table2 = table.reshape(_VOCAB * 2, _HALF)
    base = ids * 2
    idx2 = jnp.stack([base, base + 1]).reshape(_NC, _NS, _NCHUNK, _CHUNK)
    zeros = jnp.zeros((_BATCH, _HALF), jnp.float32)
    out = _sc_embed(table2, idx2, segs, zeros)
    out = out.transpose(1, 0, 2).reshape(_BATCH, _DIM)
    return jnp.expand_dims(out, axis=1)


# speedup vs baseline: 2.3548x; 1.1797x over previous
import functools

import jax
import jax.numpy as jnp
from jax import lax
from jax.experimental import pallas as pl
from jax.experimental.pallas import tpu as pltpu
from jax.experimental.pallas import tpu_sc as plsc

_VOCAB = 1000000
_DIM = 32
_BATCH = 16384
_TOTAL = 327680

_NC = 2
_NS = 16
_HALF = _DIM // 2
_CHUNK = 128
_PER_SUB = _TOTAL // _NS
_NCHUNK = _PER_SUB // _CHUNK
_SEG_PER_SUB = _BATCH // _NS
_NBUF = 4


def _sc_embed(table2, idx2, segs, zeros):
    mesh = plsc.VectorSubcoreMesh(core_axis_name="c", subcore_axis_name="s")

    @functools.partial(
        pl.kernel,
        mesh=mesh,
        compiler_params=pltpu.CompilerParams(use_tc_tiling_on_sc=False),
        out_type=jax.ShapeDtypeStruct((_NC, _BATCH, _HALF), jnp.float32),
        scratch_types=[
            pltpu.VMEM((_NCHUNK, _CHUNK), jnp.int32),
            pltpu.VMEM((_NCHUNK, _CHUNK), jnp.int32),
            [pltpu.VMEM((_CHUNK, _HALF), jnp.float32)] * _NBUF,
            pltpu.VMEM_SHARED((_BATCH, _HALF), jnp.float32),
            [pltpu.SemaphoreType.DMA] * _NBUF,
        ],
    )
    def k(table_hbm, idx_hbm, seg_hbm, zero_hbm, out_hbm,
          idx_v, seg_v, rows, acc_sh, gsem):
        c = lax.axis_index("c")
        s = lax.axis_index("s")

        pltpu.sync_copy(zero_hbm.at[pl.ds(s * _SEG_PER_SUB, _SEG_PER_SUB)],
                        acc_sh.at[pl.ds(s * _SEG_PER_SUB, _SEG_PER_SUB)])
        pltpu.sync_copy(idx_hbm.at[c, s], idx_v)
        pltpu.sync_copy(seg_hbm.at[s], seg_v)
        plsc.subcore_barrier()

        for b in range(_NBUF):
            pltpu.async_copy(table_hbm.at[idx_v.at[b]], rows[b], gsem[b])

        def group(g, carry):
            for b in range(_NBUF):
                j = g * _NBUF + b
                pltpu.make_async_copy(table_hbm.at[idx_v.at[b]],
                                      rows[b], gsem[b]).wait()
                pltpu.sync_copy(rows[b], acc_sh.at[seg_v.at[j]], add=True)
                nj = j + _NBUF

                @pl.when(nj < _NCHUNK)
                def _():
                    pltpu.async_copy(table_hbm.at[idx_v.at[nj]],
                                     rows[b], gsem[b])
            return carry

        lax.fori_loop(0, _NCHUNK // _NBUF, group, 0)
        plsc.subcore_barrier()

        pltpu.sync_copy(acc_sh.at[pl.ds(s * _SEG_PER_SUB, _SEG_PER_SUB)],
                        out_hbm.at[c, pl.ds(s * _SEG_PER_SUB, _SEG_PER_SUB)])

    return k(table2, idx2, segs, zeros)


def kernel(ids, segment_ids, table):
    ids = ids.astype(jnp.int32)
    segs = segment_ids.astype(jnp.int32).reshape(_NS, _NCHUNK, _CHUNK)
    table2 = table.reshape(_VOCAB * 2, _HALF)
    base = ids * 2
    idx2 = jnp.stack([base, base + 1]).reshape(_NC, _NS, _NCHUNK, _CHUNK)
    zeros = jnp.zeros((_BATCH, _HALF), jnp.float32)
    out = _sc_embed(table2, idx2, segs, zeros)
    out = out.transpose(1, 0, 2).reshape(_BATCH, _DIM)
    return jnp.expand_dims(out, axis=1)


# speedup vs baseline: 2.3803x; 1.0108x over previous
import functools

import jax
import jax.numpy as jnp
from jax import lax
from jax.experimental import pallas as pl
from jax.experimental.pallas import tpu as pltpu
from jax.experimental.pallas import tpu_sc as plsc

_VOCAB = 1000000
_DIM = 32
_BATCH = 16384
_TOTAL = 327680

_NC = 2
_NS = 16
_CHUNK = 128
_PER_TILE = _TOTAL // (_NC * _NS)
_NCH = _PER_TILE // _CHUNK
_NBUF = 4
_SEG_PER_SUB = _BATCH // _NS


def _sc_embed(table, ids, segs, zeros):
    mesh = plsc.VectorSubcoreMesh(core_axis_name="c", subcore_axis_name="s")

    @functools.partial(
        pl.kernel,
        mesh=mesh,
        compiler_params=pltpu.CompilerParams(use_tc_tiling_on_sc=False),
        out_type=jax.ShapeDtypeStruct((_NC, _BATCH, _DIM), jnp.float32),
        scratch_types=[
            pltpu.VMEM((_NCH, _CHUNK), jnp.int32),
            pltpu.VMEM((_NCH, _CHUNK), jnp.int32),
            [pltpu.VMEM((_CHUNK, _DIM), jnp.float32)] * _NBUF,
            pltpu.VMEM_SHARED((_BATCH, _DIM), jnp.float32),
            [pltpu.SemaphoreType.DMA] * _NBUF,
        ],
    )
    def k(table_hbm, idx_hbm, seg_hbm, zero_hbm, out_hbm,
          idx_v, seg_v, rows, acc_sh, gsem):
        c = lax.axis_index("c")
        s = lax.axis_index("s")

        pltpu.sync_copy(zero_hbm.at[pl.ds(s * _SEG_PER_SUB, _SEG_PER_SUB)],
                        acc_sh.at[pl.ds(s * _SEG_PER_SUB, _SEG_PER_SUB)])
        pltpu.sync_copy(idx_hbm.at[c, s], idx_v)
        pltpu.sync_copy(seg_hbm.at[c, s], seg_v)
        plsc.subcore_barrier()

        for b in range(_NBUF):
            pltpu.async_copy(table_hbm.at[idx_v.at[b]], rows[b], gsem[b])

        def group(g, carry):
            for b in range(_NBUF):
                j = g * _NBUF + b
                pltpu.make_async_copy(table_hbm.at[idx_v.at[b]],
                                      rows[b], gsem[b]).wait()
                pltpu.sync_copy(rows[b], acc_sh.at[seg_v.at[j]], add=True)
                nj = j + _NBUF

                @pl.when(nj < _NCH)
                def _():
                    pltpu.async_copy(table_hbm.at[idx_v.at[nj]],
                                     rows[b], gsem[b])
            return carry

        lax.fori_loop(0, _NCH // _NBUF, group, 0)
        plsc.subcore_barrier()

        pltpu.sync_copy(acc_sh.at[pl.ds(s * _SEG_PER_SUB, _SEG_PER_SUB)],
                        out_hbm.at[c, pl.ds(s * _SEG_PER_SUB, _SEG_PER_SUB)])

    return k(table, ids, segs, zeros)


def _tc_combine(parts):
    def body(p_ref, o_ref):
        o_ref[...] = p_ref[0] + p_ref[1]

    return pl.pallas_call(
        body,
        out_shape=jax.ShapeDtypeStruct((_BATCH, _DIM), jnp.float32),
    )(parts)


def kernel(ids, segment_ids, table):
    ids = ids.astype(jnp.int32).reshape(_NC, _NS, _NCH, _CHUNK)
    segs = segment_ids.astype(jnp.int32).reshape(_NC, _NS, _NCH, _CHUNK)
    zeros = jnp.zeros((_BATCH, _DIM), jnp.float32)
    parts = _sc_embed(table, ids, segs, zeros)
    out = _tc_combine(parts)
    return out[:, None, :]


# speedup vs baseline: 2.6318x; 1.1057x over previous
import functools

import jax
import jax.numpy as jnp
from jax import lax
from jax.experimental import pallas as pl
from jax.experimental.pallas import tpu as pltpu
from jax.experimental.pallas import tpu_sc as plsc

_VOCAB = 1000000
_DIM = 32
_BATCH = 16384
_TOTAL = 327680

_NC = 2
_NS = 16
_CHUNK = 128
_PER_TILE = _TOTAL // (_NC * _NS)
_NCH = _PER_TILE // _CHUNK
_NBUF = 4
_SEG_PER_SUB = _BATCH // _NS


def _sc_embed(table, ids, segs, zeros):
    mesh = plsc.VectorSubcoreMesh(core_axis_name="c", subcore_axis_name="s")

    @functools.partial(
        pl.kernel,
        mesh=mesh,
        compiler_params=pltpu.CompilerParams(use_tc_tiling_on_sc=False),
        out_type=jax.ShapeDtypeStruct((_NC, _BATCH, _DIM), jnp.float32),
        scratch_types=[
            pltpu.VMEM((_NCH, _CHUNK), jnp.int32),
            pltpu.VMEM((_NCH, _CHUNK), jnp.int32),
            [pltpu.VMEM((_CHUNK, _DIM), jnp.float32)] * _NBUF,
            pltpu.VMEM_SHARED((_BATCH, _DIM), jnp.float32),
            [pltpu.SemaphoreType.DMA] * _NBUF,
        ],
    )
    def k(table_hbm, idx_hbm, seg_hbm, zero_hbm, out_hbm,
          idx_v, seg_v, rows, acc_sh, gsem):
        c = lax.axis_index("c")
        s = lax.axis_index("s")

        pltpu.sync_copy(zero_hbm.at[pl.ds(s * _SEG_PER_SUB, _SEG_PER_SUB)],
                        acc_sh.at[pl.ds(s * _SEG_PER_SUB, _SEG_PER_SUB)])
        pltpu.sync_copy(idx_hbm.at[c, s], idx_v)
        pltpu.sync_copy(seg_hbm.at[c, s], seg_v)
        plsc.subcore_barrier()

        for b in range(_NBUF):
            pltpu.async_copy(table_hbm.at[idx_v.at[b]], rows[b], gsem[b])

        def group(g, carry):
            for b in range(_NBUF):
                j = g * _NBUF + b
                pltpu.make_async_copy(table_hbm.at[idx_v.at[b]],
                                      rows[b], gsem[b]).wait()
                pltpu.sync_copy(rows[b], acc_sh.at[seg_v.at[j]], add=True)
                nj = j + _NBUF

                @pl.when(nj < _NCH)
                def _():
                    pltpu.async_copy(table_hbm.at[idx_v.at[nj]],
                                     rows[b], gsem[b])
            return carry

        lax.fori_loop(0, _NCH // _NBUF, group, 0)
        plsc.subcore_barrier()

        pltpu.sync_copy(acc_sh.at[pl.ds(s * _SEG_PER_SUB, _SEG_PER_SUB)],
                        out_hbm.at[c, pl.ds(s * _SEG_PER_SUB, _SEG_PER_SUB)])

    return k(table, ids, segs, zeros)


_PACK_C = 2048


def _tc_pack(table_t):
    grid = (1000000 + _PACK_C - 1) // _PACK_C

    def body(x_ref, y_ref):
        x = x_ref[...]
        t = x.T.reshape(_PACK_C // 8, 8, _DIM)
        y_ref[:, :, 0:_DIM] = t

    return pl.pallas_call(
        body,
        grid=(grid,),
        in_specs=[pl.BlockSpec((_DIM, _PACK_C), lambda g: (0, g))],
        out_specs=pl.BlockSpec((_PACK_C // 8, 8, 128), lambda g: (g, 0, 0)),
        out_shape=jax.ShapeDtypeStruct((125000, 8, 128), jnp.float32),
    )(table_t)


def _tc_combine(parts):
    def body(p_ref, o_ref):
        o_ref[...] = p_ref[0] + p_ref[1]

    return pl.pallas_call(
        body,
        out_shape=jax.ShapeDtypeStruct((_BATCH, _DIM), jnp.float32),
    )(parts)


def kernel(ids, segment_ids, table):
    ids = (ids.astype(jnp.int32) * 4).reshape(_NC, _NS, _NCH, _CHUNK)
    segs = segment_ids.astype(jnp.int32).reshape(_NC, _NS, _NCH, _CHUNK)
    zeros = jnp.zeros((_BATCH, _DIM), jnp.float32)
    packed = _tc_pack(table.T).reshape(4 * _VOCAB, _DIM)
    parts = _sc_embed(packed, ids, segs, zeros)
    out = _tc_combine(parts)
    return out[:, None, :]


# speedup vs baseline: 3.1276x; 1.1884x over previous
import functools

import jax
import jax.numpy as jnp
from jax import lax
from jax.experimental import pallas as pl
from jax.experimental.pallas import tpu as pltpu
from jax.experimental.pallas import tpu_sc as plsc

_VOCAB = 1000000
_DIM = 32
_BATCH = 16384
_TOTAL = 327680

_NC = 2
_NS = 16
_CHUNK = 128
_PER_TILE = _TOTAL // (_NC * _NS)
_NCH = _PER_TILE // _CHUNK
_NBUF = 4
_SEG_PER_SUB = _BATCH // _NS


def _sc_embed(table, ids, segs, zeros):
    mesh = plsc.VectorSubcoreMesh(core_axis_name="c", subcore_axis_name="s")

    @functools.partial(
        pl.kernel,
        mesh=mesh,
        compiler_params=pltpu.CompilerParams(use_tc_tiling_on_sc=False),
        out_type=jax.ShapeDtypeStruct((_NC, _BATCH, _DIM), jnp.float32),
        scratch_types=[
            pltpu.VMEM((_NCH, _CHUNK), jnp.int32),
            pltpu.VMEM((_NCH, _CHUNK), jnp.int32),
            [pltpu.VMEM((_CHUNK, _DIM), jnp.float32)] * _NBUF,
            pltpu.VMEM_SHARED((_BATCH, _DIM), jnp.float32),
            [pltpu.SemaphoreType.DMA] * _NBUF,
        ],
    )
    def k(table_hbm, idx_hbm, seg_hbm, zero_hbm, out_hbm,
          idx_v, seg_v, rows, acc_sh, gsem):
        c = lax.axis_index("c")
        s = lax.axis_index("s")

        pltpu.sync_copy(zero_hbm.at[pl.ds(s * _SEG_PER_SUB, _SEG_PER_SUB)],
                        acc_sh.at[pl.ds(s * _SEG_PER_SUB, _SEG_PER_SUB)])
        pltpu.sync_copy(idx_hbm.at[c, s], idx_v)
        pltpu.sync_copy(seg_hbm.at[c, s], seg_v)
        plsc.subcore_barrier()

        for b in range(_NBUF):
            pltpu.async_copy(table_hbm.at[idx_v.at[b]], rows[b], gsem[b])

        def group(g, carry):
            for b in range(_NBUF):
                j = g * _NBUF + b
                pltpu.make_async_copy(table_hbm.at[idx_v.at[b]],
                                      rows[b], gsem[b]).wait()
                pltpu.sync_copy(rows[b], acc_sh.at[seg_v.at[j]], add=True)
                nj = j + _NBUF

                @pl.when(nj < _NCH)
                def _():
                    pltpu.async_copy(table_hbm.at[idx_v.at[nj]],
                                     rows[b], gsem[b])
            return carry

        lax.fori_loop(0, _NCH // _NBUF, group, 0)
        plsc.subcore_barrier()

        pltpu.sync_copy(acc_sh.at[pl.ds(s * _SEG_PER_SUB, _SEG_PER_SUB)],
                        out_hbm.at[c, pl.ds(s * _SEG_PER_SUB, _SEG_PER_SUB)])

    return k(table, ids, segs, zeros)


_PACK_R = 2048


def _tc_pack(table_t):
    grid = (250000 + _PACK_R - 1) // _PACK_R

    def body(x_ref, y_ref):
        x = x_ref[...]
        t3 = x.T.reshape(_PACK_R, 4, _DIM)
        for a in range(4):
            y_ref[:, 32 * a:32 * (a + 1)] = t3[:, a, :]

    return pl.pallas_call(
        body,
        grid=(grid,),
        in_specs=[pl.BlockSpec((_DIM, 4 * _PACK_R), lambda g: (0, g))],
        out_specs=pl.BlockSpec((_PACK_R, 128), lambda g: (g, 0)),
        out_shape=jax.ShapeDtypeStruct((250000, 128), jnp.float32),
    )(table_t)


def _tc_combine(parts):
    def body(p_ref, o_ref):
        o_ref[...] = p_ref[0] + p_ref[1]

    return pl.pallas_call(
        body,
        out_shape=jax.ShapeDtypeStruct((_BATCH, _DIM), jnp.float32),
    )(parts)


def kernel(ids, segment_ids, table):
    ids = ids.astype(jnp.int32).reshape(_NC, _NS, _NCH, _CHUNK)
    segs = segment_ids.astype(jnp.int32).reshape(_NC, _NS, _NCH, _CHUNK)
    zeros = jnp.zeros((_BATCH, _DIM), jnp.float32)
    packed = _tc_pack(table.T).reshape(_VOCAB, _DIM)
    parts = _sc_embed(packed, ids, segs, zeros)
    out = _tc_combine(parts)
    return out[:, None, :]


# speedup vs baseline: 4.1250x; 1.3189x over previous
import functools

import jax
import jax.numpy as jnp
from jax import lax
from jax.experimental import pallas as pl
from jax.experimental.pallas import tpu as pltpu
from jax.experimental.pallas import tpu_sc as plsc

_VOCAB = 1000000
_DIM = 32
_BATCH = 16384
_TOTAL = 327680

_NC = 2
_NS = 16
_CHUNK = 128
_PER_TILE = _TOTAL // (_NC * _NS)
_NCH = _PER_TILE // _CHUNK
_NBUF = 4
_SEG_PER_SUB = _BATCH // _NS


def _sc_embed(table, ids, segs, zeros):
    mesh = plsc.VectorSubcoreMesh(core_axis_name="c", subcore_axis_name="s")

    @functools.partial(
        pl.kernel,
        mesh=mesh,
        compiler_params=pltpu.CompilerParams(use_tc_tiling_on_sc=False),
        out_type=jax.ShapeDtypeStruct((_NC, _BATCH, _DIM), jnp.float32),
        scratch_types=[
            pltpu.VMEM((_NCH, _CHUNK), jnp.int32),
            pltpu.VMEM((_NCH, _CHUNK), jnp.int32),
            [pltpu.VMEM((_CHUNK, _DIM), jnp.float32)] * _NBUF,
            pltpu.VMEM_SHARED((_BATCH, _DIM), jnp.float32),
            [pltpu.SemaphoreType.DMA] * _NBUF,
        ],
    )
    def k(table_hbm, idx_hbm, seg_hbm, zero_hbm, out_hbm,
          idx_v, seg_v, rows, acc_sh, gsem):
        c = lax.axis_index("c")
        s = lax.axis_index("s")

        pltpu.sync_copy(zero_hbm.at[pl.ds(s * _SEG_PER_SUB, _SEG_PER_SUB)],
                        acc_sh.at[pl.ds(s * _SEG_PER_SUB, _SEG_PER_SUB)])
        pltpu.sync_copy(idx_hbm.at[c, s], idx_v)
        pltpu.sync_copy(seg_hbm.at[c, s], seg_v)
        plsc.subcore_barrier()

        for b in range(_NBUF):
            pltpu.async_copy(table_hbm.at[idx_v.at[b]], rows[b], gsem[b])

        def group(g, carry):
            for b in range(_NBUF):
                j = g * _NBUF + b
                pltpu.make_async_copy(table_hbm.at[idx_v.at[b]],
                                      rows[b], gsem[b]).wait()
                pltpu.sync_copy(rows[b], acc_sh.at[seg_v.at[j]], add=True)
                nj = j + _NBUF

                @pl.when(nj < _NCH)
                def _():
                    pltpu.async_copy(table_hbm.at[idx_v.at[nj]],
                                     rows[b], gsem[b])
            return carry

        lax.fori_loop(0, _NCH // _NBUF, group, 0)
        plsc.subcore_barrier()

        pltpu.sync_copy(acc_sh.at[pl.ds(s * _SEG_PER_SUB, _SEG_PER_SUB)],
                        out_hbm.at[c, pl.ds(s * _SEG_PER_SUB, _SEG_PER_SUB)])

    return k(table, ids, segs, zeros)


_PACK_R = 2048
_PACK_G = (1000000 + 4 * _PACK_R - 1) // (4 * _PACK_R)


def _tc_pack(table_t):
    def body(x_ref, y_ref):
        x = x_ref[...]
        t4 = x.T.reshape(4, _PACK_R, _DIM)
        for a in range(4):
            y_ref[:, 32 * a:32 * (a + 1)] = t4[a]

    return pl.pallas_call(
        body,
        grid=(_PACK_G,),
        in_specs=[pl.BlockSpec((_DIM, 4 * _PACK_R), lambda g: (0, g))],
        out_specs=pl.BlockSpec((_PACK_R, 128), lambda g: (g, 0)),
        out_shape=jax.ShapeDtypeStruct((_PACK_G * _PACK_R, 128), jnp.float32),
    )(table_t)


def _tc_combine(parts):
    def body(p_ref, o_ref):
        o_ref[...] = p_ref[0] + p_ref[1]

    return pl.pallas_call(
        body,
        out_shape=jax.ShapeDtypeStruct((_BATCH, _DIM), jnp.float32),
    )(parts)


def kernel(ids, segment_ids, table):
    i32 = ids.astype(jnp.int32)
    slot = ((i32 & ~jnp.int32(8191)) | ((i32 & 2047) << 2) | ((i32 >> 11) & 3))
    ids = slot.reshape(_NC, _NS, _NCH, _CHUNK)
    segs = segment_ids.astype(jnp.int32).reshape(_NC, _NS, _NCH, _CHUNK)
    zeros = jnp.zeros((_BATCH, _DIM), jnp.float32)
    packed = _tc_pack(table.T).reshape(_PACK_G * _PACK_R * 4, _DIM)
    parts = _sc_embed(packed, ids, segs, zeros)
    out = _tc_combine(parts)
    return out[:, None, :]


# speedup vs baseline: 6.2571x; 1.5169x over previous
import functools

import jax
import jax.numpy as jnp
from jax import lax
from jax.experimental import pallas as pl
from jax.experimental.pallas import tpu as pltpu
from jax.experimental.pallas import tpu_sc as plsc

_VOCAB = 1000000
_DIM = 32
_BATCH = 16384
_TOTAL = 327680

_NC = 2
_NS = 16
_CHUNK = 128
_PER_TILE = _TOTAL // (_NC * _NS)
_NCH = _PER_TILE // _CHUNK
_NBUF = 4
_SEG_PER_SUB = _BATCH // _NS


def _sc_embed(table, ids, segs, zeros):
    mesh = plsc.VectorSubcoreMesh(core_axis_name="c", subcore_axis_name="s")

    @functools.partial(
        pl.kernel,
        mesh=mesh,
        compiler_params=pltpu.CompilerParams(use_tc_tiling_on_sc=False),
        out_type=jax.ShapeDtypeStruct((_NC, _BATCH, _DIM), jnp.float32),
        scratch_types=[
            pltpu.VMEM((_NCH, _CHUNK), jnp.int32),
            pltpu.VMEM((_NCH, _CHUNK), jnp.int32),
            [pltpu.VMEM((_CHUNK, _DIM), jnp.float32)] * _NBUF,
            pltpu.VMEM_SHARED((_BATCH, _DIM), jnp.float32),
            [pltpu.SemaphoreType.DMA] * _NBUF,
        ],
    )
    def k(table_hbm, idx_hbm, seg_hbm, zero_hbm, out_hbm,
          idx_v, seg_v, rows, acc_sh, gsem):
        c = lax.axis_index("c")
        s = lax.axis_index("s")

        pltpu.sync_copy(zero_hbm.at[pl.ds(s * _SEG_PER_SUB, _SEG_PER_SUB)],
                        acc_sh.at[pl.ds(s * _SEG_PER_SUB, _SEG_PER_SUB)])
        pltpu.sync_copy(idx_hbm.at[c, s], idx_v)
        pltpu.sync_copy(seg_hbm.at[c, s], seg_v)
        plsc.subcore_barrier()

        for b in range(_NBUF):
            pltpu.async_copy(table_hbm.at[idx_v.at[b]], rows[b], gsem[b])

        def group(g, carry):
            for b in range(_NBUF):
                j = g * _NBUF + b
                pltpu.make_async_copy(table_hbm.at[idx_v.at[b]],
                                      rows[b], gsem[b]).wait()
                pltpu.sync_copy(rows[b], acc_sh.at[seg_v.at[j]], add=True)
                nj = j + _NBUF

                @pl.when(nj < _NCH)
                def _():
                    pltpu.async_copy(table_hbm.at[idx_v.at[nj]],
                                     rows[b], gsem[b])
            return carry

        lax.fori_loop(0, _NCH // _NBUF, group, 0)
        plsc.subcore_barrier()

        pltpu.sync_copy(acc_sh.at[pl.ds(s * _SEG_PER_SUB, _SEG_PER_SUB)],
                        out_hbm.at[c, pl.ds(s * _SEG_PER_SUB, _SEG_PER_SUB)])

    return k(table, ids, segs, zeros)


_PACK_R = 2048
_PACK_G = (1000000 + 4 * _PACK_R - 1) // (4 * _PACK_R)


def _tc_pack(table_t):
    def body(x_ref, y_ref):
        z = jnp.concatenate(
            [x_ref[:, _PACK_R * a:_PACK_R * (a + 1)] for a in range(4)],
            axis=0)
        y_ref[...] = z.T

    return pl.pallas_call(
        body,
        grid=(_PACK_G,),
        in_specs=[pl.BlockSpec((_DIM, 4 * _PACK_R), lambda g: (0, g))],
        out_specs=pl.BlockSpec((_PACK_R, 128), lambda g: (g, 0)),
        out_shape=jax.ShapeDtypeStruct((_PACK_G * _PACK_R, 128), jnp.float32),
    )(table_t)


def _tc_combine(parts):
    def body(p_ref, o_ref):
        o_ref[...] = p_ref[0] + p_ref[1]

    return pl.pallas_call(
        body,
        out_shape=jax.ShapeDtypeStruct((_BATCH, _DIM), jnp.float32),
    )(parts)


def kernel(ids, segment_ids, table):
    i32 = ids.astype(jnp.int32)
    slot = ((i32 & ~jnp.int32(8191)) | ((i32 & 2047) << 2) | ((i32 >> 11) & 3))
    ids = slot.reshape(_NC, _NS, _NCH, _CHUNK)
    segs = segment_ids.astype(jnp.int32).reshape(_NC, _NS, _NCH, _CHUNK)
    zeros = jnp.zeros((_BATCH, _DIM), jnp.float32)
    packed = _tc_pack(table.T).reshape(_PACK_G * _PACK_R * 4, _DIM)
    parts = _sc_embed(packed, ids, segs, zeros)
    out = _tc_combine(parts)
    return out[:, None, :]
